# baseline (speedup 1.0000x reference)
import jax
import jax.numpy as jnp
from jax import lax
from jax.experimental import pallas as pl
from jax.experimental.pallas import tpu as pltpu
from jax.experimental.pallas import tpu_sc as plsc

N = 10000
M = 10000
E = 320000
D_IN = 128
HID = 128
OUT = 64
HH = HID // 2

NC = 2
NS = 16
NW = NC * NS
EPW = E // NW
K = 80
NCHUNK = EPW // K
NBUF = 12
NROUND = -(-NCHUNK // NBUF)
RCHUNK = 80
DPAD = 10240
COLS = DPAD // NS


def _sc_seg_sum(seg_rows, ntab, with_deg=False):
  mesh = plsc.VectorSubcoreMesh(core_axis_name="c", subcore_axis_name="s")
  ch_total = seg_rows // RCHUNK
  ch_iters = -(-ch_total // NS)
  d = HH

  out_type = [jax.ShapeDtypeStruct((NC, seg_rows, d), jnp.float32)] * ntab
  if with_deg:
    out_type += [jax.ShapeDtypeStruct((NC, DPAD, 16), jnp.float32)] * 2

  scratch = [
      pltpu.VMEM((NCHUNK, K), jnp.int32),
      pltpu.VMEM((NCHUNK, K), jnp.int32),
      [pltpu.VMEM((K, d), jnp.float32) for _ in range(NBUF)],
      pltpu.VMEM((RCHUNK, d), jnp.float32),
      pltpu.VMEM_SHARED((seg_rows, d), jnp.float32),
      [pltpu.SemaphoreType.DMA for _ in range(NBUF)],
      pltpu.SemaphoreType.DMA,
  ]
  if with_deg:
    scratch += [
        pltpu.VMEM((K, 16), jnp.float32),
        pltpu.VMEM((COLS, 16), jnp.float32),
        pltpu.VMEM_SHARED((DPAD, 16), jnp.float32),
        pltpu.VMEM_SHARED((DPAD, 16), jnp.float32),
        pltpu.SemaphoreType.DMA,
    ]

  def body(*refs):
    tbls = refs[:ntab]
    gidx3, sidx3 = refs[ntab], refs[ntab + 1]
    nout = 2 * ntab + 2 + (2 if with_deg else 0)
    acc_os = refs[ntab + 2:ntab + 2 + ntab]
    if with_deg:
      de_o, dv_o = refs[2 * ntab + 2:nout]
      (gi_all, si_all, bufs, zbuf_v, acc_sh, sems, sem,
       ones_v, dstage_v, dege_sh, degv_sh, sem_d) = refs[nout:]
    else:
      gi_all, si_all, bufs, zbuf_v, acc_sh, sems, sem = refs[nout:]
    c = lax.axis_index("c")
    s = lax.axis_index("s")
    wid = c * NS + s
    zeros16 = jnp.zeros((16,), jnp.float32)
    ones16 = jnp.ones((16,), jnp.float32)

    pltpu.sync_copy(gidx3.at[wid], gi_all)
    pltpu.sync_copy(sidx3.at[wid], si_all)

    def zrow(r, carry):
      for cc in range(d // 16):
        zbuf_v[r, pl.ds(cc * 16, 16)] = zeros16
      return carry
    lax.fori_loop(0, RCHUNK, zrow, 0)

    if with_deg:
      def fill(r, carry):
        ones_v[r, pl.ds(0, 16)] = ones16
        return carry
      lax.fori_loop(0, K, fill, 0)

      def fill0(r, carry):
        dstage_v[r, pl.ds(0, 16)] = zeros16
        return carry
      lax.fori_loop(0, COLS, fill0, 0)
      pltpu.sync_copy(dstage_v, dege_sh.at[pl.ds(s * COLS, COLS)])
      pltpu.sync_copy(dstage_v, degv_sh.at[pl.ds(s * COLS, COLS)])

    for ti, (tbl, acc_o) in enumerate(zip(tbls, acc_os)):
      def zacc(i, carry):
        ch = s + i * NS

        @pl.when(ch < ch_total)
        def _():
          pltpu.sync_copy(zbuf_v, acc_sh.at[pl.ds(ch * RCHUNK, RCHUNK)])
        return carry
      lax.fori_loop(0, ch_iters, zacc, 0)

      plsc.subcore_barrier()

      for b in range(NBUF):
        pltpu.async_copy(tbl.at[gi_all.at[b]], bufs[b], sems[b])

      def round_(r, carry):
        for b in range(NBUF):
          j = r * NBUF + b

          @pl.when(j < NCHUNK)
          def _():
            pltpu.make_async_copy(tbl.at[gi_all.at[j]], bufs[b],
                                  sems[b]).wait()
            pltpu.sync_copy(bufs[b], acc_sh.at[si_all.at[j]], add=True)

            @pl.when(j + NBUF < NCHUNK)
            def _():
              pltpu.async_copy(tbl.at[gi_all.at[j + NBUF]], bufs[b], sems[b])
        return carry
      lax.fori_loop(0, NROUND, round_, 0)

      if with_deg and ti == 0:
        def dchunk(j, carry):
          pltpu.async_copy(ones_v, dege_sh.at[si_all.at[j]], sem_d, add=True)
          pltpu.async_copy(ones_v, degv_sh.at[gi_all.at[j]], sem_d, add=True)
          return carry
        lax.fori_loop(0, NCHUNK, dchunk, 0)

      if with_deg and ti == ntab - 1:
        def ddrain(j, carry):
          pltpu.make_async_copy(ones_v, dege_sh.at[si_all.at[0]],
                                sem_d).wait()
          pltpu.make_async_copy(ones_v, degv_sh.at[gi_all.at[0]],
                                sem_d).wait()
          return carry
        lax.fori_loop(0, NCHUNK, ddrain, 0)

      plsc.subcore_barrier()

      def dump(i, carry):
        ch = s + i * NS

        @pl.when(ch < ch_total)
        def _():
          start = ch * RCHUNK
          pltpu.sync_copy(acc_sh.at[pl.ds(start, RCHUNK)], zbuf_v)
          pltpu.sync_copy(zbuf_v, acc_o.at[c, pl.ds(start, RCHUNK)])
        return carry
      lax.fori_loop(0, ch_iters, dump, 0)

      lax.fori_loop(0, RCHUNK, zrow, 0)

    if with_deg:
      for sh, out_ref in ((dege_sh, de_o), (degv_sh, dv_o)):
        pltpu.sync_copy(sh.at[pl.ds(s * COLS, COLS)], dstage_v)
        pltpu.sync_copy(dstage_v, out_ref.at[c, pl.ds(s * COLS, COLS)])

  return pl.kernel(body, out_type=tuple(out_type), mesh=mesh,
                   scratch_types=tuple(scratch),
                   compiler_params=pltpu.CompilerParams(
                       use_tc_tiling_on_sc=False))


def _sc_degrees():
  mesh = plsc.VectorSubcoreMesh(core_axis_name="c", subcore_axis_name="s")
  out_type = (jax.ShapeDtypeStruct((NC, DPAD, 16), jnp.float32),) * 2
  scratch = [
      pltpu.VMEM((NCHUNK, K), jnp.int32),
      pltpu.VMEM((NCHUNK, K), jnp.int32),
      pltpu.VMEM((K, 16), jnp.float32),
      pltpu.VMEM((COLS, 16), jnp.float32),
      pltpu.VMEM_SHARED((DPAD, 16), jnp.float32),
      pltpu.VMEM_SHARED((DPAD, 16), jnp.float32),
      pltpu.SemaphoreType.DMA,
  ]

  def body(gidx3, sidx3, de_o, dv_o, gi_all, si_all, ones_v, dstage_v,
           dege_sh, degv_sh, sem):
    c = lax.axis_index("c")
    s = lax.axis_index("s")
    wid = c * NS + s
    zeros16 = jnp.zeros((16,), jnp.float32)
    ones16 = jnp.ones((16,), jnp.float32)

    pltpu.sync_copy(gidx3.at[wid], gi_all)
    pltpu.sync_copy(sidx3.at[wid], si_all)

    def fill(r, carry):
      ones_v[r, pl.ds(0, 16)] = ones16
      return carry
    lax.fori_loop(0, K, fill, 0)

    def fill0(r, carry):
      dstage_v[r, pl.ds(0, 16)] = zeros16
      return carry
    lax.fori_loop(0, COLS, fill0, 0)

    pltpu.sync_copy(dstage_v, dege_sh.at[pl.ds(s * COLS, COLS)])
    pltpu.sync_copy(dstage_v, degv_sh.at[pl.ds(s * COLS, COLS)])
    plsc.subcore_barrier()

    def chunk(j, carry):
      pltpu.async_copy(ones_v, dege_sh.at[si_all.at[j]], sem, add=True)
      pltpu.async_copy(ones_v, degv_sh.at[gi_all.at[j]], sem, add=True)
      return carry
    lax.fori_loop(0, NCHUNK, chunk, 0)

    def drain(j, carry):
      pltpu.make_async_copy(ones_v, dege_sh.at[si_all.at[0]], sem).wait()
      pltpu.make_async_copy(ones_v, degv_sh.at[gi_all.at[0]], sem).wait()
      return carry
    lax.fori_loop(0, NCHUNK, drain, 0)

    plsc.subcore_barrier()
    for sh, out_ref in ((dege_sh, de_o), (degv_sh, dv_o)):
      pltpu.sync_copy(sh.at[pl.ds(s * COLS, COLS)], dstage_v)
      pltpu.sync_copy(dstage_v, out_ref.at[c, pl.ds(s * COLS, COLS)])

  return pl.kernel(body, out_type=out_type, mesh=mesh,
                   scratch_types=tuple(scratch),
                   compiler_params=pltpu.CompilerParams(
                       use_tc_tiling_on_sc=False))


_sc_pass2 = _sc_seg_sum(M, 2)
_sc_pass1 = _sc_seg_sum(M, 1)
_sc_deg = _sc_degrees()

_BS = 1000


def _mm_split(x, w):
  n, din = x.shape

  def body(x_ref, w_ref, o0_ref, o1_ref):
    t = jnp.dot(x_ref[...], w_ref[...], preferred_element_type=jnp.float32)
    o0_ref[...] = t[:, :HH]
    o1_ref[...] = t[:, HH:]

  shp = jax.ShapeDtypeStruct((n, HH), jnp.float32)
  return pl.pallas_call(
      body,
      grid=(n // _BS,),
      in_specs=[pl.BlockSpec((_BS, din), lambda i: (i, 0)),
                pl.BlockSpec((din, HID), lambda i: (0, 0))],
      out_specs=[pl.BlockSpec((_BS, HH), lambda i: (i, 0))] * 2,
      out_shape=(shp, shp),
  )(x, w)


def _comb2(a0, a1, inv_col):
  n = a0.shape[1]

  def body(a0_ref, a1_ref, i_ref, o0_ref, o1_ref):
    inv = 1.0 / jnp.maximum(i_ref[0] + i_ref[1], 1.0)
    o0_ref[...] = (a0_ref[0] + a0_ref[1]) * inv
    o1_ref[...] = (a1_ref[0] + a1_ref[1]) * inv

  shp = jax.ShapeDtypeStruct((n, HH), jnp.float32)
  return pl.pallas_call(
      body,
      grid=(n // _BS,),
      in_specs=[pl.BlockSpec((NC, _BS, HH), lambda i: (0, i, 0)),
                pl.BlockSpec((NC, _BS, HH), lambda i: (0, i, 0)),
                pl.BlockSpec((NC, _BS, 1), lambda i: (0, i, 0))],
      out_specs=[pl.BlockSpec((_BS, HH), lambda i: (i, 0))] * 2,
      out_shape=(shp, shp),
  )(a0, a1, inv_col)


def _elu_norm_mm(b0, b1, inv_col, bias_row, w):
  n = b0.shape[1]
  dout = w.shape[1]

  def body(b0_ref, b1_ref, i_ref, bias_ref, w_ref, o_ref):
    inv = 1.0 / jnp.maximum(i_ref[0] + i_ref[1], 1.0)
    t0 = (b0_ref[0] + b0_ref[1]) * inv + bias_ref[:, :HH]
    t1 = (b1_ref[0] + b1_ref[1]) * inv + bias_ref[:, HH:]
    t = jnp.concatenate([t0, t1], axis=1)
    h = jnp.where(t > 0.0, t, jnp.exp(t) - 1.0)
    o_ref[...] = jnp.dot(h, w_ref[...], preferred_element_type=jnp.float32)

  return pl.pallas_call(
      body,
      grid=(n // _BS,),
      in_specs=[pl.BlockSpec((NC, _BS, HH), lambda i: (0, i, 0)),
                pl.BlockSpec((NC, _BS, HH), lambda i: (0, i, 0)),
                pl.BlockSpec((NC, _BS, 1), lambda i: (0, i, 0)),
                pl.BlockSpec((1, HID), lambda i: (0, 0)),
                pl.BlockSpec((HID, dout), lambda i: (0, 0))],
      out_specs=pl.BlockSpec((_BS, dout), lambda i: (i, 0)),
      out_shape=jax.ShapeDtypeStruct((n, dout), jnp.float32),
  )(b0, b1, inv_col, bias_row, w)


def _comb_scale(acc, inv_col, d):
  n = acc.shape[1]

  def body(a_ref, i_ref, o_ref):
    inv = 1.0 / jnp.maximum(i_ref[0] + i_ref[1], 1.0)
    o_ref[...] = (a_ref[0] + a_ref[1]) * inv

  return pl.pallas_call(
      body,
      grid=(n // _BS,),
      in_specs=[pl.BlockSpec((NC, _BS, d), lambda i: (0, i, 0)),
                pl.BlockSpec((NC, _BS, 1), lambda i: (0, i, 0))],
      out_specs=pl.BlockSpec((_BS, d), lambda i: (i, 0)),
      out_shape=jax.ShapeDtypeStruct((n, d), jnp.float32),
  )(acc, inv_col)


def _final(acc, inv_col, bias_row, d):
  n = acc.shape[1]

  def body(a_ref, i_ref, b_ref, o_ref):
    inv = 1.0 / jnp.maximum(i_ref[0] + i_ref[1], 1.0)
    o_ref[...] = (a_ref[0] + a_ref[1]) * inv + b_ref[...]

  return pl.pallas_call(
      body,
      grid=(n // _BS,),
      in_specs=[pl.BlockSpec((NC, _BS, d), lambda i: (0, i, 0)),
                pl.BlockSpec((NC, _BS, 1), lambda i: (0, i, 0)),
                pl.BlockSpec((1, d), lambda i: (0, 0))],
      out_specs=pl.BlockSpec((_BS, d), lambda i: (i, 0)),
      out_shape=jax.ShapeDtypeStruct((n, d), jnp.float32),
  )(acc, inv_col, bias_row)


def kernel(x, edge_index, edge_weight, W1, b1, W2, b2):
  node3 = edge_index[0].reshape(NW, NCHUNK, K)
  he3 = edge_index[1].reshape(NW, NCHUNK, K)

  xv0, xv1 = _mm_split(x, W1)
  de_p, dv_p = _sc_deg(node3, he3)
  a0, a1 = _sc_pass2(xv0, xv1, node3, he3)
  dcol_e = de_p[:, :M, :1]
  dcol_v = dv_p[:, :N, :1]

  xe10, xe11 = _comb2(a0, a1, dcol_e)
  b0, b1_acc = _sc_pass2(xe10, xe11, he3, node3)
  xv2 = _elu_norm_mm(b0, b1_acc, dcol_v, b1.reshape(1, HID), W2)
  (c_acc,) = _sc_pass1(xv2, node3, he3)
  xe2 = _comb_scale(c_acc, dcol_e, OUT)
  (d_acc,) = _sc_pass1(xe2, he3, node3)
  return _final(d_acc, dcol_v, b2.reshape(1, OUT), OUT)


# speedup vs baseline: 1.0424x; 1.0424x over previous
import jax
import jax.numpy as jnp
from jax import lax
from jax.experimental import pallas as pl
from jax.experimental.pallas import tpu as pltpu
from jax.experimental.pallas import tpu_sc as plsc

N = 10000
M = 10000
E = 320000
D_IN = 128
HID = 128
OUT = 64

NC = 2
NS = 16
EPT = E // NS
K = 80
NCH = EPT // K
NBUF = 8
NROUND = -(-NCH // NBUF)
RCHUNK = 80
DPAD = 10240
COLS = DPAD // NS


def _sc_layer(d, apply_elu):
  mesh = plsc.VectorSubcoreMesh(core_axis_name="c", subcore_axis_name="s")
  ch_total = M // RCHUNK
  ch_iters = -(-ch_total // NS)

  out_type = (jax.ShapeDtypeStruct((NC * M, d), jnp.float32),
              jax.ShapeDtypeStruct((NC * N, d), jnp.float32))

  scratch = [
      pltpu.VMEM((NCH, K), jnp.int32),
      pltpu.VMEM((NCH, K), jnp.int32),
      [pltpu.VMEM((K, d), jnp.float32) for _ in range(NBUF)],
      pltpu.VMEM((RCHUNK, d), jnp.float32),
      pltpu.VMEM((RCHUNK, 16), jnp.float32),
      pltpu.VMEM((d,), jnp.float32),
      pltpu.VMEM_SHARED((M, d), jnp.float32),
      [pltpu.SemaphoreType.DMA for _ in range(NBUF)],
  ]

  def body(tbl, gidx3, sidx3, dege, degv, bias2, xe_o, out_o,
           gi_all, si_all, bufs, zbuf_v, dbuf_v, bias_v, acc_sh, sems):
    c = lax.axis_index("c")
    s = lax.axis_index("s")
    zeros16 = jnp.zeros((16,), jnp.float32)

    pltpu.sync_copy(gidx3.at[s], gi_all)
    pltpu.sync_copy(sidx3.at[s], si_all)
    pltpu.sync_copy(bias2.at[c], bias_v)

    def adjust(ref, off):
      def arow(r, carry):
        for t in range(K // 16):
          ref[r, pl.ds(t * 16, 16)] = ref[r, pl.ds(t * 16, 16)] + off
        return carry
      lax.fori_loop(0, NCH, arow, 0)

    adjust(gi_all, c * N)

    def zrow(r, carry):
      for cc in range(d // 16):
        zbuf_v[r, pl.ds(cc * 16, 16)] = zeros16
      return carry
    lax.fori_loop(0, RCHUNK, zrow, 0)

    def zacc(i, carry):
      ch = s + i * NS

      @pl.when(ch < ch_total)
      def _():
        pltpu.sync_copy(zbuf_v, acc_sh.at[pl.ds(ch * RCHUNK, RCHUNK)])
      return carry
    lax.fori_loop(0, ch_iters, zacc, 0)

    plsc.subcore_barrier()

    def run_stage(gtbl, g_all, s_all):
      for b in range(NBUF):
        pltpu.async_copy(gtbl.at[g_all.at[b]], bufs[b], sems[b])

      def round_(r, carry):
        for b in range(NBUF):
          j = r * NBUF + b

          @pl.when(j < NCH)
          def _():
            pltpu.make_async_copy(gtbl.at[g_all.at[j]], bufs[b],
                                  sems[b]).wait()
            pltpu.sync_copy(bufs[b], acc_sh.at[s_all.at[j]], add=True)

            @pl.when(j + NBUF < NCH)
            def _():
              pltpu.async_copy(gtbl.at[g_all.at[j + NBUF]], bufs[b], sems[b])
        return carry
      lax.fori_loop(0, NROUND, round_, 0)

    def dump_norm(deg, dst, dst_base, with_act):
      def dump(i, carry):
        ch = s + i * NS

        @pl.when(ch < ch_total)
        def _():
          start = ch * RCHUNK
          pltpu.sync_copy(acc_sh.at[pl.ds(start, RCHUNK)], zbuf_v)
          pltpu.sync_copy(deg.at[pl.ds(start, RCHUNK)], dbuf_v)

          def nrow(r, carry2):
            inv = 1.0 / jnp.maximum(dbuf_v[r, pl.ds(0, 16)], 1.0)
            for cc in range(d // 16):
              t = zbuf_v[r, pl.ds(cc * 16, 16)] * inv
              if with_act:
                t = t + bias_v[pl.ds(cc * 16, 16)]
                if apply_elu:
                  t = jnp.where(t > 0.0, t, jnp.exp(t) - 1.0)
              zbuf_v[r, pl.ds(cc * 16, 16)] = t
            return carry2
          lax.fori_loop(0, RCHUNK, nrow, 0)
          pltpu.sync_copy(zbuf_v, dst.at[pl.ds(dst_base + start, RCHUNK)])
        return carry
      lax.fori_loop(0, ch_iters, dump, 0)

    run_stage(tbl, gi_all, si_all)
    plsc.subcore_barrier()
    dump_norm(dege, xe_o, c * M, False)
    lax.fori_loop(0, RCHUNK, zrow, 0)
    adjust(si_all, c * M)

    def zacc2(i, carry):
      ch = s + i * NS

      @pl.when(ch < ch_total)
      def _():
        pltpu.sync_copy(zbuf_v, acc_sh.at[pl.ds(ch * RCHUNK, RCHUNK)])
      return carry
    lax.fori_loop(0, ch_iters, zacc2, 0)
    adjust(gi_all, -(c * N))
    plsc.subcore_barrier()

    run_stage(xe_o, si_all, gi_all)
    plsc.subcore_barrier()
    dump_norm(degv, out_o, c * N, True)

  return pl.kernel(body, out_type=out_type, mesh=mesh,
                   scratch_types=tuple(scratch),
                   compiler_params=pltpu.CompilerParams(
                       use_tc_tiling_on_sc=False))


def _sc_degrees():
  mesh = plsc.VectorSubcoreMesh(core_axis_name="c", subcore_axis_name="s")
  out_type = (jax.ShapeDtypeStruct((DPAD, 16), jnp.float32),) * 2
  scratch = [
      pltpu.VMEM((NCH, K), jnp.int32),
      pltpu.VMEM((K, 16), jnp.float32),
      pltpu.VMEM((COLS, 16), jnp.float32),
      pltpu.VMEM_SHARED((DPAD, 16), jnp.float32),
      pltpu.SemaphoreType.DMA,
  ]

  def body(gidx3, sidx3, de_o, dv_o, idx_all, ones_v, dstage_v, sh, sem):
    c = lax.axis_index("c")
    s = lax.axis_index("s")
    zeros16 = jnp.zeros((16,), jnp.float32)
    ones16 = jnp.ones((16,), jnp.float32)

    @pl.when(c == 0)
    def _():
      pltpu.sync_copy(sidx3.at[s], idx_all)

    @pl.when(c == 1)
    def _():
      pltpu.sync_copy(gidx3.at[s], idx_all)

    def fill(r, carry):
      ones_v[r, pl.ds(0, 16)] = ones16
      return carry
    lax.fori_loop(0, K, fill, 0)

    def fill0(r, carry):
      dstage_v[r, pl.ds(0, 16)] = zeros16
      return carry
    lax.fori_loop(0, COLS, fill0, 0)

    pltpu.sync_copy(dstage_v, sh.at[pl.ds(s * COLS, COLS)])
    plsc.subcore_barrier()

    def chunk(j, carry):
      pltpu.async_copy(ones_v, sh.at[idx_all.at[j]], sem, add=True)
      return carry
    lax.fori_loop(0, NCH, chunk, 0)

    def drain(j, carry):
      pltpu.make_async_copy(ones_v, sh.at[idx_all.at[0]], sem).wait()
      return carry
    lax.fori_loop(0, NCH, drain, 0)

    plsc.subcore_barrier()
    pltpu.sync_copy(sh.at[pl.ds(s * COLS, COLS)], dstage_v)

    @pl.when(c == 0)
    def _():
      pltpu.sync_copy(dstage_v, de_o.at[pl.ds(s * COLS, COLS)])

    @pl.when(c == 1)
    def _():
      pltpu.sync_copy(dstage_v, dv_o.at[pl.ds(s * COLS, COLS)])

  return pl.kernel(body, out_type=out_type, mesh=mesh,
                   scratch_types=tuple(scratch),
                   compiler_params=pltpu.CompilerParams(
                       use_tc_tiling_on_sc=False))


_sc_layer1 = _sc_layer(HID // 2, True)
_sc_layer2 = _sc_layer(OUT // 2, False)
_sc_deg = _sc_degrees()

_BS = 1000


def _mm_halves(x, ws, dh):
  n, din = x.shape

  def body(x_ref, w_ref, o_ref):
    o_ref[...] = jnp.dot(x_ref[...], w_ref[0],
                         preferred_element_type=jnp.float32)

  return pl.pallas_call(
      body,
      grid=(NC, n // _BS),
      in_specs=[pl.BlockSpec((_BS, din), lambda h, i: (i, 0)),
                pl.BlockSpec((1, din, dh), lambda h, i: (h, 0, 0))],
      out_specs=pl.BlockSpec((_BS, dh), lambda h, i: (h * (n // _BS) + i, 0)),
      out_shape=jax.ShapeDtypeStruct((NC * n, dh), jnp.float32),
  )(x, ws)


def _mm2_halves(h_s, ws, dh):
  n2, dhin = h_s.shape
  n = n2 // NC

  def body(h0_ref, h1_ref, w_ref, o_ref):
    h = jnp.concatenate([h0_ref[...], h1_ref[...]], axis=1)
    o_ref[...] = jnp.dot(h, w_ref[0], preferred_element_type=jnp.float32)

  return pl.pallas_call(
      body,
      grid=(NC, n // _BS),
      in_specs=[pl.BlockSpec((_BS, dhin), lambda h, i: (i, 0)),
                pl.BlockSpec((_BS, dhin), lambda h, i: ((n // _BS) + i, 0)),
                pl.BlockSpec((1, 2 * dhin, dh), lambda h, i: (h, 0, 0))],
      out_specs=pl.BlockSpec((_BS, dh), lambda h, i: (h * (n // _BS) + i, 0)),
      out_shape=jax.ShapeDtypeStruct((NC * n, dh), jnp.float32),
  )(h_s, h_s, ws)


def _concat_halves(o_s, dh):
  n = o_s.shape[0] // NC

  def body(a_ref, b_ref, o_ref):
    o_ref[:, :dh] = a_ref[...]
    o_ref[:, dh:] = b_ref[...]

  return pl.pallas_call(
      body,
      grid=(n // _BS,),
      in_specs=[pl.BlockSpec((_BS, dh), lambda i: (i, 0)),
                pl.BlockSpec((_BS, dh), lambda i: ((n // _BS) + i, 0))],
      out_specs=pl.BlockSpec((_BS, NC * dh), lambda i: (i, 0)),
      out_shape=jax.ShapeDtypeStruct((n, NC * dh), jnp.float32),
  )(o_s, o_s)


def kernel(x, edge_index, edge_weight, W1, b1, W2, b2):
  node3 = edge_index[0].reshape(NS, NCH, K)
  he3 = edge_index[1].reshape(NS, NCH, K)

  de, dv = _sc_deg(node3, he3)
  w1s = jnp.stack([W1[:, :HID // 2], W1[:, HID // 2:]])
  w2s = jnp.stack([W2[:, :OUT // 2], W2[:, OUT // 2:]])
  xv_s = _mm_halves(x, w1s, HID // 2)
  _, h_s = _sc_layer1(xv_s, node3, he3, de, dv, b1.reshape(NC, HID // 2))
  xv2_s = _mm2_halves(h_s, w2s, OUT // 2)
  _, o_s = _sc_layer2(xv2_s, node3, he3, de, dv, b2.reshape(NC, OUT // 2))
  return _concat_halves(o_s, OUT // 2)
